# baseline (speedup 1.0000x reference)
import functools

import jax
import jax.numpy as jnp
from jax import lax
from jax.experimental import pallas as pl
from jax.experimental.pallas import tpu as pltpu
from jax.experimental.pallas import tpu_sc as plsc

ALPHA = 0.1
THETA = 0.5
EPS = 1e-5

ROWS_BLK = 1000
CHUNK = 128
NS = 16
HP = lax.Precision.HIGHEST


def _linear_body(x_ref, w_ref, b_ref, za_ref, zb_ref, xsa_ref, xsb_ref):
    hd = za_ref.shape[1]
    z = jnp.dot(x_ref[...], w_ref[...], preferred_element_type=jnp.float32,
                precision=HP)
    z = z + b_ref[...]
    za_ref[...] = z[:, :hd]
    zb_ref[...] = z[:, hd:]
    xs = (ALPHA / (1.0 - ALPHA)) * z
    xsa_ref[...] = xs[:, :hd]
    xsb_ref[...] = xs[:, hd:]


def _make_linear(n, d, r):
    nb = n // r
    hd = d // 2
    return pl.pallas_call(
        _linear_body,
        grid=(nb,),
        in_specs=[
            pl.BlockSpec((r, d), lambda j: (j, 0)),
            pl.BlockSpec((d, d), lambda j: (0, 0)),
            pl.BlockSpec((1, d), lambda j: (0, 0)),
        ],
        out_specs=[
            pl.BlockSpec((r, hd), lambda j: (j, 0)),
            pl.BlockSpec((r, hd), lambda j: (j, 0)),
            pl.BlockSpec((r, hd), lambda j: (j, 0)),
            pl.BlockSpec((r, hd), lambda j: (j, 0)),
        ],
        out_shape=[jax.ShapeDtypeStruct((n, hd), jnp.float32)] * 4,
    )


def _dense_bn_body(nb, n_total, h_a, h_b, w_ref, g_ref, be_ref,
                   za_ref, zb_ref, G_s, s_s, aff_s):
    p = pl.program_id(0)
    j = pl.program_id(1)
    hd = za_ref.shape[1]
    hsum = jnp.concatenate([h_a[...], h_b[...]], axis=1)

    @pl.when(jnp.logical_and(p == 0, j == 0))
    def _():
        G_s[...] = jnp.zeros_like(G_s)
        s_s[...] = jnp.zeros_like(s_s)

    @pl.when(p == 0)
    def _():
        G_s[...] += lax.dot_general(hsum, hsum, (((0,), (0,)), ((), ())),
                                    preferred_element_type=jnp.float32,
                                    precision=HP)
        s_s[...] += jnp.sum(hsum, axis=0, keepdims=True)

        @pl.when(j == nb - 1)
        def _():
            W = w_ref[...]
            colsum_z = jnp.dot(s_s[...], W,
                               preferred_element_type=jnp.float32, precision=HP)
            mean = colsum_z / n_total
            GW = jnp.dot(G_s[...], W,
                         preferred_element_type=jnp.float32, precision=HP)
            ez2 = jnp.sum(W * GW, axis=0, keepdims=True) / n_total
            var = ez2 - mean * mean
            scale = g_ref[...] * lax.rsqrt(var + EPS)
            aff_s[0:1, :] = scale
            aff_s[1:2, :] = be_ref[...] - mean * scale

    @pl.when(p == 1)
    def _():
        z = jnp.dot(hsum, w_ref[...], preferred_element_type=jnp.float32,
                    precision=HP)
        zn = jnp.maximum(z * aff_s[0:1, :] + aff_s[1:2, :], 0.0)
        za_ref[...] = zn[:, :hd]
        zb_ref[...] = zn[:, hd:]


def _make_dense_bn(n, d, r):
    nb = n // r
    hd = d // 2
    return pl.pallas_call(
        functools.partial(_dense_bn_body, nb, float(n)),
        grid=(2, nb),
        in_specs=[
            pl.BlockSpec((r, hd), lambda p, j: (j, 0)),
            pl.BlockSpec((r, hd), lambda p, j: (j, 0)),
            pl.BlockSpec((d, d), lambda p, j: (0, 0)),
            pl.BlockSpec((1, d), lambda p, j: (0, 0)),
            pl.BlockSpec((1, d), lambda p, j: (0, 0)),
        ],
        out_specs=[
            pl.BlockSpec((r, hd), lambda p, j: (j, 0)),
            pl.BlockSpec((r, hd), lambda p, j: (j, 0)),
        ],
        out_shape=[jax.ShapeDtypeStruct((n, hd), jnp.float32)] * 2,
        scratch_shapes=[
            pltpu.VMEM((d, d), jnp.float32),
            pltpu.VMEM((1, d), jnp.float32),
            pltpu.VMEM((2, d), jnp.float32),
        ],
    )


def _dense_final_body(h_a, h_b, w_ref, z_ref):
    hsum = jnp.concatenate([h_a[...], h_b[...]], axis=1)
    z_ref[...] = jnp.dot(hsum, w_ref[...], preferred_element_type=jnp.float32,
                         precision=HP)


def _make_dense_final(n, d, r):
    nb = n // r
    hd = d // 2
    return pl.pallas_call(
        _dense_final_body,
        grid=(nb,),
        in_specs=[
            pl.BlockSpec((r, hd), lambda j: (j, 0)),
            pl.BlockSpec((r, hd), lambda j: (j, 0)),
            pl.BlockSpec((d, d), lambda j: (0, 0)),
        ],
        out_specs=pl.BlockSpec((r, d), lambda j: (j, 0)),
        out_shape=jax.ShapeDtypeStruct((n, d), jnp.float32),
    )


def _segsum_body(z_a, z_b, x_a, x_b, srcp, dstp, out_a, out_b,
                 acc, idx_s, idx_d, rows, sem):
    cid = lax.axis_index("c")
    sid = lax.axis_index("s")
    nc = idx_s.shape[0]
    n_out = out_a.shape[0]
    copy_per = 8 * (n_out // (NS * 8))
    copy_last = n_out - copy_per * (NS - 1)
    init_per = copy_per
    init_last = acc.shape[0] - init_per * (NS - 1)

    ib = sid * init_per
    last = sid == NS - 1

    def _init(x_half):
        @pl.when(jnp.logical_not(last))
        def _():
            pltpu.sync_copy(x_half.at[pl.ds(ib, init_per)],
                            acc.at[pl.ds(ib, init_per)])

        @pl.when(last)
        def _():
            pltpu.sync_copy(x_half.at[pl.ds(ib, copy_last)],
                            acc.at[pl.ds(ib, copy_last)])
            pltpu.sync_copy(x_half.at[pl.ds(0, init_last - copy_last)],
                            acc.at[pl.ds(ib + copy_last,
                                         init_last - copy_last)])

    @pl.when(cid == 0)
    def _():
        _init(x_a)

    @pl.when(cid == 1)
    def _():
        _init(x_b)

    pltpu.sync_copy(srcp.at[sid], idx_s)
    pltpu.sync_copy(dstp.at[sid], idx_d)
    plsc.subcore_barrier()

    def _run(table):
        def body(jj, carry):
            pltpu.async_copy(table.at[idx_s.at[jj]], rows, sem).wait()
            pltpu.sync_copy(rows, acc.at[idx_d.at[jj]], add=True)
            return carry
        lax.fori_loop(0, nc, body, 0)

    @pl.when(cid == 0)
    def _():
        _run(z_a)

    @pl.when(cid == 1)
    def _():
        _run(z_b)

    plsc.subcore_barrier()

    ob = sid * copy_per

    @pl.when(jnp.logical_and(cid == 0, jnp.logical_not(last)))
    def _():
        pltpu.sync_copy(acc.at[pl.ds(ob, copy_per)],
                        out_a.at[pl.ds(ob, copy_per)])

    @pl.when(jnp.logical_and(cid == 0, last))
    def _():
        pltpu.sync_copy(acc.at[pl.ds(ob, copy_last)],
                        out_a.at[pl.ds(ob, copy_last)])

    @pl.when(jnp.logical_and(cid == 1, jnp.logical_not(last)))
    def _():
        pltpu.sync_copy(acc.at[pl.ds(ob, copy_per)],
                        out_b.at[pl.ds(ob, copy_per)])

    @pl.when(jnp.logical_and(cid == 1, last))
    def _():
        pltpu.sync_copy(acc.at[pl.ds(ob, copy_last)],
                        out_b.at[pl.ds(ob, copy_last)])


def _make_segsum(n, d, nc):
    hd = d // 2
    nacc = NS * 8 * (-(-(n + 1) // (NS * 8)))
    mesh = plsc.VectorSubcoreMesh(core_axis_name="c", subcore_axis_name="s")
    return pl.kernel(
        _segsum_body,
        out_type=[jax.ShapeDtypeStruct((n, hd), jnp.float32)] * 2,
        mesh=mesh,
        scratch_types=[
            pltpu.VMEM_SHARED((nacc, hd), jnp.float32),
            pltpu.VMEM((nc, CHUNK), jnp.int32),
            pltpu.VMEM((nc, CHUNK), jnp.int32),
            pltpu.VMEM((CHUNK, hd), jnp.float32),
            pltpu.SemaphoreType.DMA,
        ],
    )


def kernel(x, edge_index, lin_W, lin_b, conv_W, bn_gamma, bn_beta):
    import numpy as np
    n, d = x.shape
    e = edge_index.shape[1]
    nlayers = conv_W.shape[0]

    betas = [float(np.log(THETA / (l + 1) + 1.0)) for l in range(nlayers)]
    eye = jnp.eye(d, dtype=jnp.float32)
    bb = jnp.asarray(betas, jnp.float32).reshape(nlayers, 1, 1)
    Wp = (1.0 - ALPHA) * ((1.0 - bb) * eye[None] + bb * conv_W)

    per = -(-e // NS)
    nc = -(-per // CHUNK)
    pk = 1 << 14
    key = jnp.sort(edge_index[0] * pk + edge_index[1])
    src = jnp.pad(key // pk, (0, NS * per - e))
    dst = jnp.pad(key % pk, (0, NS * per - e), constant_values=n)
    srcp = jnp.pad(src.reshape(NS, per), ((0, 0), (0, nc * CHUNK - per)))
    dstp = jnp.pad(dst.reshape(NS, per), ((0, 0), (0, nc * CHUNK - per)),
                   constant_values=n).reshape(NS, nc, CHUNK)
    srcp = srcp.reshape(NS, nc, CHUNK)

    lin = _make_linear(n, d, ROWS_BLK)
    dense_bn = _make_dense_bn(n, d, ROWS_BLK)
    dense_final = _make_dense_final(n, d, ROWS_BLK)
    segsum = _make_segsum(n, d, nc)

    x0_a, x0_b, xs_a, xs_b = lin(x, lin_W, lin_b.reshape(1, d))
    za, zb = x0_a, x0_b
    for l in range(nlayers):
        h_a, h_b = segsum(za, zb, xs_a, xs_b, srcp, dstp)
        if l < nlayers - 1:
            za, zb = dense_bn(h_a, h_b, Wp[l],
                              bn_gamma[l].reshape(1, d),
                              bn_beta[l].reshape(1, d))
        else:
            z = dense_final(h_a, h_b, Wp[l])
    return z


# speedup vs baseline: 1.4066x; 1.4066x over previous
import functools

import jax
import jax.numpy as jnp
from jax import lax
from jax.experimental import pallas as pl
from jax.experimental.pallas import tpu as pltpu
from jax.experimental.pallas import tpu_sc as plsc

ALPHA = 0.1
THETA = 0.5
EPS = 1e-5

ROWS_BLK = 1000
CHUNK = 128
NS = 16
HP = lax.Precision.HIGHEST


def _linear_body(x_ref, w_ref, b_ref, za_ref, zb_ref, xsa_ref, xsb_ref):
    hd = za_ref.shape[1]
    z = jnp.dot(x_ref[...], w_ref[...], preferred_element_type=jnp.float32,
                precision=HP)
    z = z + b_ref[...]
    za_ref[...] = z[:, :hd]
    zb_ref[...] = z[:, hd:]
    xs = (ALPHA / (1.0 - ALPHA)) * z
    xsa_ref[...] = xs[:, :hd]
    xsb_ref[...] = xs[:, hd:]


def _make_linear(n, d, r):
    nb = n // r
    hd = d // 2
    return pl.pallas_call(
        _linear_body,
        grid=(nb,),
        in_specs=[
            pl.BlockSpec((r, d), lambda j: (j, 0)),
            pl.BlockSpec((d, d), lambda j: (0, 0)),
            pl.BlockSpec((1, d), lambda j: (0, 0)),
        ],
        out_specs=[
            pl.BlockSpec((r, hd), lambda j: (j, 0)),
            pl.BlockSpec((r, hd), lambda j: (j, 0)),
            pl.BlockSpec((r, hd), lambda j: (j, 0)),
            pl.BlockSpec((r, hd), lambda j: (j, 0)),
        ],
        out_shape=[jax.ShapeDtypeStruct((n, hd), jnp.float32)] * 4,
    )


def _dense_bn_body(nb, n_total, h_a, h_b, w_ref, g_ref, be_ref,
                   za_ref, zb_ref, G_s, s_s, aff_s):
    p = pl.program_id(0)
    j = pl.program_id(1)
    hd = za_ref.shape[1]
    hsum = jnp.concatenate([h_a[...], h_b[...]], axis=1)

    @pl.when(jnp.logical_and(p == 0, j == 0))
    def _():
        G_s[...] = jnp.zeros_like(G_s)
        s_s[...] = jnp.zeros_like(s_s)

    @pl.when(p == 0)
    def _():
        G_s[...] += lax.dot_general(hsum, hsum, (((0,), (0,)), ((), ())),
                                    preferred_element_type=jnp.float32,
                                    precision=HP)
        s_s[...] += jnp.sum(hsum, axis=0, keepdims=True)

        @pl.when(j == nb - 1)
        def _():
            W = w_ref[...]
            colsum_z = jnp.dot(s_s[...], W,
                               preferred_element_type=jnp.float32, precision=HP)
            mean = colsum_z / n_total
            GW = jnp.dot(G_s[...], W,
                         preferred_element_type=jnp.float32, precision=HP)
            ez2 = jnp.sum(W * GW, axis=0, keepdims=True) / n_total
            var = ez2 - mean * mean
            scale = g_ref[...] * lax.rsqrt(var + EPS)
            aff_s[0:1, :] = scale
            aff_s[1:2, :] = be_ref[...] - mean * scale

    @pl.when(p == 1)
    def _():
        z = jnp.dot(hsum, w_ref[...], preferred_element_type=jnp.float32,
                    precision=HP)
        zn = jnp.maximum(z * aff_s[0:1, :] + aff_s[1:2, :], 0.0)
        za_ref[...] = zn[:, :hd]
        zb_ref[...] = zn[:, hd:]


def _make_dense_bn(n, d, r):
    nb = n // r
    hd = d // 2
    return pl.pallas_call(
        functools.partial(_dense_bn_body, nb, float(n)),
        grid=(2, nb),
        in_specs=[
            pl.BlockSpec((r, hd), lambda p, j: (j, 0)),
            pl.BlockSpec((r, hd), lambda p, j: (j, 0)),
            pl.BlockSpec((d, d), lambda p, j: (0, 0)),
            pl.BlockSpec((1, d), lambda p, j: (0, 0)),
            pl.BlockSpec((1, d), lambda p, j: (0, 0)),
        ],
        out_specs=[
            pl.BlockSpec((r, hd), lambda p, j: (j, 0)),
            pl.BlockSpec((r, hd), lambda p, j: (j, 0)),
        ],
        out_shape=[jax.ShapeDtypeStruct((n, hd), jnp.float32)] * 2,
        scratch_shapes=[
            pltpu.VMEM((d, d), jnp.float32),
            pltpu.VMEM((1, d), jnp.float32),
            pltpu.VMEM((2, d), jnp.float32),
        ],
    )


def _dense_final_body(h_a, h_b, w_ref, z_ref):
    hsum = jnp.concatenate([h_a[...], h_b[...]], axis=1)
    z_ref[...] = jnp.dot(hsum, w_ref[...], preferred_element_type=jnp.float32,
                         precision=HP)


def _make_dense_final(n, d, r):
    nb = n // r
    hd = d // 2
    return pl.pallas_call(
        _dense_final_body,
        grid=(nb,),
        in_specs=[
            pl.BlockSpec((r, hd), lambda j: (j, 0)),
            pl.BlockSpec((r, hd), lambda j: (j, 0)),
            pl.BlockSpec((d, d), lambda j: (0, 0)),
        ],
        out_specs=pl.BlockSpec((r, d), lambda j: (j, 0)),
        out_shape=jax.ShapeDtypeStruct((n, d), jnp.float32),
    )


def _segsum_body(z_a, z_b, x_a, x_b, srcp, dstp, out_a, out_b,
                 acc, idx_s, idx_d, rows, sem):
    cid = lax.axis_index("c")
    sid = lax.axis_index("s")
    nc = idx_s.shape[0]
    n_out = out_a.shape[0]
    copy_per = 8 * (n_out // (NS * 8))
    copy_last = n_out - copy_per * (NS - 1)
    init_per = copy_per
    init_last = acc.shape[0] - init_per * (NS - 1)

    ib = sid * init_per
    last = sid == NS - 1

    def _init(x_half):
        @pl.when(jnp.logical_not(last))
        def _():
            pltpu.sync_copy(x_half.at[pl.ds(ib, init_per)],
                            acc.at[pl.ds(ib, init_per)])

        @pl.when(last)
        def _():
            pltpu.sync_copy(x_half.at[pl.ds(ib, copy_last)],
                            acc.at[pl.ds(ib, copy_last)])
            pltpu.sync_copy(x_half.at[pl.ds(0, init_last - copy_last)],
                            acc.at[pl.ds(ib + copy_last,
                                         init_last - copy_last)])

    @pl.when(cid == 0)
    def _():
        _init(x_a)

    @pl.when(cid == 1)
    def _():
        _init(x_b)

    pltpu.sync_copy(srcp.at[sid], idx_s)
    pltpu.sync_copy(dstp.at[sid], idx_d)
    plsc.subcore_barrier()

    def _run(table):
        def body(jj, carry):
            pltpu.async_copy(table.at[idx_s.at[jj]], rows, sem).wait()
            pltpu.sync_copy(rows, acc.at[idx_d.at[jj]], add=True)
            return carry
        lax.fori_loop(0, nc, body, 0)

    @pl.when(cid == 0)
    def _():
        _run(z_a)

    @pl.when(cid == 1)
    def _():
        _run(z_b)

    plsc.subcore_barrier()

    ob = sid * copy_per

    @pl.when(jnp.logical_and(cid == 0, jnp.logical_not(last)))
    def _():
        pltpu.sync_copy(acc.at[pl.ds(ob, copy_per)],
                        out_a.at[pl.ds(ob, copy_per)])

    @pl.when(jnp.logical_and(cid == 0, last))
    def _():
        pltpu.sync_copy(acc.at[pl.ds(ob, copy_last)],
                        out_a.at[pl.ds(ob, copy_last)])

    @pl.when(jnp.logical_and(cid == 1, jnp.logical_not(last)))
    def _():
        pltpu.sync_copy(acc.at[pl.ds(ob, copy_per)],
                        out_b.at[pl.ds(ob, copy_per)])

    @pl.when(jnp.logical_and(cid == 1, last))
    def _():
        pltpu.sync_copy(acc.at[pl.ds(ob, copy_last)],
                        out_b.at[pl.ds(ob, copy_last)])


def _make_segsum(n, d, nc):
    hd = d // 2
    nacc = NS * 8 * (-(-(n + 1) // (NS * 8)))
    mesh = plsc.VectorSubcoreMesh(core_axis_name="c", subcore_axis_name="s")
    return pl.kernel(
        _segsum_body,
        out_type=[jax.ShapeDtypeStruct((n, hd), jnp.float32)] * 2,
        mesh=mesh,
        scratch_types=[
            pltpu.VMEM_SHARED((nacc, hd), jnp.float32),
            pltpu.VMEM((nc, CHUNK), jnp.int32),
            pltpu.VMEM((nc, CHUNK), jnp.int32),
            pltpu.VMEM((CHUNK, hd), jnp.float32),
            pltpu.SemaphoreType.DMA,
        ],
    )


def kernel(x, edge_index, lin_W, lin_b, conv_W, bn_gamma, bn_beta):
    import numpy as np
    n, d = x.shape
    e = edge_index.shape[1]
    nlayers = conv_W.shape[0]

    betas = [float(np.log(THETA / (l + 1) + 1.0)) for l in range(nlayers)]
    eye = jnp.eye(d, dtype=jnp.float32)
    bb = jnp.asarray(betas, jnp.float32).reshape(nlayers, 1, 1)
    Wp = (1.0 - ALPHA) * ((1.0 - bb) * eye[None] + bb * conv_W)

    per = -(-e // NS)
    nc = -(-per // CHUNK)
    src = jnp.pad(edge_index[0], (0, NS * per - e))
    dst = jnp.pad(edge_index[1], (0, NS * per - e), constant_values=n)
    srcp = jnp.pad(src.reshape(NS, per), ((0, 0), (0, nc * CHUNK - per)))
    dstp = jnp.pad(dst.reshape(NS, per), ((0, 0), (0, nc * CHUNK - per)),
                   constant_values=n).reshape(NS, nc, CHUNK)
    srcp = srcp.reshape(NS, nc, CHUNK)

    lin = _make_linear(n, d, ROWS_BLK)
    dense_bn = _make_dense_bn(n, d, ROWS_BLK)
    dense_final = _make_dense_final(n, d, ROWS_BLK)
    segsum = _make_segsum(n, d, nc)

    x0_a, x0_b, xs_a, xs_b = lin(x, lin_W, lin_b.reshape(1, d))
    za, zb = x0_a, x0_b
    for l in range(nlayers):
        h_a, h_b = segsum(za, zb, xs_a, xs_b, srcp, dstp)
        if l < nlayers - 1:
            za, zb = dense_bn(h_a, h_b, Wp[l],
                              bn_gamma[l].reshape(1, d),
                              bn_beta[l].reshape(1, d))
        else:
            z = dense_final(h_a, h_b, Wp[l])
    return z


# speedup vs baseline: 1.4312x; 1.0175x over previous
import functools

import jax
import jax.numpy as jnp
from jax import lax
from jax.experimental import pallas as pl
from jax.experimental.pallas import tpu as pltpu
from jax.experimental.pallas import tpu_sc as plsc

ALPHA = 0.1
THETA = 0.5
EPS = 1e-5

ROWS_BLK = 2000
CHUNK = 128
NS = 16
HP = lax.Precision.HIGHEST


def _linear_body(x_ref, w_ref, b_ref, za_ref, zb_ref, xsa_ref, xsb_ref):
    hd = za_ref.shape[1]
    z = jnp.dot(x_ref[...], w_ref[...], preferred_element_type=jnp.float32,
                precision=HP)
    z = z + b_ref[...]
    za_ref[...] = z[:, :hd]
    zb_ref[...] = z[:, hd:]
    xs = (ALPHA / (1.0 - ALPHA)) * z
    xsa_ref[...] = xs[:, :hd]
    xsb_ref[...] = xs[:, hd:]


def _make_linear(n, d, r):
    nb = n // r
    hd = d // 2
    return pl.pallas_call(
        _linear_body,
        grid=(nb,),
        in_specs=[
            pl.BlockSpec((r, d), lambda j: (j, 0)),
            pl.BlockSpec((d, d), lambda j: (0, 0)),
            pl.BlockSpec((1, d), lambda j: (0, 0)),
        ],
        out_specs=[
            pl.BlockSpec((r, hd), lambda j: (j, 0)),
            pl.BlockSpec((r, hd), lambda j: (j, 0)),
            pl.BlockSpec((r, hd), lambda j: (j, 0)),
            pl.BlockSpec((r, hd), lambda j: (j, 0)),
        ],
        out_shape=[jax.ShapeDtypeStruct((n, hd), jnp.float32)] * 4,
    )


def _dense_bn_body(nb, n_total, h_a, h_b, w_ref, g_ref, be_ref,
                   za_ref, zb_ref, G_s, s_s, aff_s):
    p = pl.program_id(0)
    j = pl.program_id(1)
    hd = za_ref.shape[1]
    hsum = jnp.concatenate([h_a[...], h_b[...]], axis=1)

    @pl.when(jnp.logical_and(p == 0, j == 0))
    def _():
        G_s[...] = jnp.zeros_like(G_s)
        s_s[...] = jnp.zeros_like(s_s)

    @pl.when(p == 0)
    def _():
        G_s[...] += lax.dot_general(hsum, hsum, (((0,), (0,)), ((), ())),
                                    preferred_element_type=jnp.float32,
                                    precision=HP)
        s_s[...] += jnp.sum(hsum, axis=0, keepdims=True)

        @pl.when(j == nb - 1)
        def _():
            W = w_ref[...]
            colsum_z = jnp.dot(s_s[...], W,
                               preferred_element_type=jnp.float32, precision=HP)
            mean = colsum_z / n_total
            GW = jnp.dot(G_s[...], W,
                         preferred_element_type=jnp.float32, precision=HP)
            ez2 = jnp.sum(W * GW, axis=0, keepdims=True) / n_total
            var = ez2 - mean * mean
            scale = g_ref[...] * lax.rsqrt(var + EPS)
            aff_s[0:1, :] = scale
            aff_s[1:2, :] = be_ref[...] - mean * scale

    @pl.when(p == 1)
    def _():
        z = jnp.dot(hsum, w_ref[...], preferred_element_type=jnp.float32,
                    precision=HP)
        zn = jnp.maximum(z * aff_s[0:1, :] + aff_s[1:2, :], 0.0)
        za_ref[...] = zn[:, :hd]
        zb_ref[...] = zn[:, hd:]


def _make_dense_bn(n, d, r):
    nb = n // r
    hd = d // 2
    return pl.pallas_call(
        functools.partial(_dense_bn_body, nb, float(n)),
        grid=(2, nb),
        in_specs=[
            pl.BlockSpec((r, hd), lambda p, j: (j, 0)),
            pl.BlockSpec((r, hd), lambda p, j: (j, 0)),
            pl.BlockSpec((d, d), lambda p, j: (0, 0)),
            pl.BlockSpec((1, d), lambda p, j: (0, 0)),
            pl.BlockSpec((1, d), lambda p, j: (0, 0)),
        ],
        out_specs=[
            pl.BlockSpec((r, hd), lambda p, j: (j, 0)),
            pl.BlockSpec((r, hd), lambda p, j: (j, 0)),
        ],
        out_shape=[jax.ShapeDtypeStruct((n, hd), jnp.float32)] * 2,
        scratch_shapes=[
            pltpu.VMEM((d, d), jnp.float32),
            pltpu.VMEM((1, d), jnp.float32),
            pltpu.VMEM((2, d), jnp.float32),
        ],
    )


def _dense_final_body(h_a, h_b, w_ref, z_ref):
    hsum = jnp.concatenate([h_a[...], h_b[...]], axis=1)
    z_ref[...] = jnp.dot(hsum, w_ref[...], preferred_element_type=jnp.float32,
                         precision=HP)


def _make_dense_final(n, d, r):
    nb = n // r
    hd = d // 2
    return pl.pallas_call(
        _dense_final_body,
        grid=(nb,),
        in_specs=[
            pl.BlockSpec((r, hd), lambda j: (j, 0)),
            pl.BlockSpec((r, hd), lambda j: (j, 0)),
            pl.BlockSpec((d, d), lambda j: (0, 0)),
        ],
        out_specs=pl.BlockSpec((r, d), lambda j: (j, 0)),
        out_shape=jax.ShapeDtypeStruct((n, d), jnp.float32),
    )


def _segsum_body(z_a, z_b, x_a, x_b, srcp, dstp, out_a, out_b,
                 acc, idx_s, idx_d, rows, sem):
    cid = lax.axis_index("c")
    sid = lax.axis_index("s")
    nc = idx_s.shape[0]
    n_out = out_a.shape[0]
    copy_per = 8 * (n_out // (NS * 8))
    copy_last = n_out - copy_per * (NS - 1)
    init_per = copy_per
    init_last = acc.shape[0] - init_per * (NS - 1)

    ib = sid * init_per
    last = sid == NS - 1

    def _init(x_half):
        @pl.when(jnp.logical_not(last))
        def _():
            pltpu.sync_copy(x_half.at[pl.ds(ib, init_per)],
                            acc.at[pl.ds(ib, init_per)])

        @pl.when(last)
        def _():
            pltpu.sync_copy(x_half.at[pl.ds(ib, copy_last)],
                            acc.at[pl.ds(ib, copy_last)])
            pltpu.sync_copy(x_half.at[pl.ds(0, init_last - copy_last)],
                            acc.at[pl.ds(ib + copy_last,
                                         init_last - copy_last)])

    @pl.when(cid == 0)
    def _():
        _init(x_a)

    @pl.when(cid == 1)
    def _():
        _init(x_b)

    pltpu.sync_copy(srcp.at[sid], idx_s)
    pltpu.sync_copy(dstp.at[sid], idx_d)
    plsc.subcore_barrier()

    def _run(table):
        def body(jj, carry):
            pltpu.async_copy(table.at[idx_s.at[jj]], rows, sem).wait()
            pltpu.sync_copy(rows, acc.at[idx_d.at[jj]], add=True)
            return carry
        lax.fori_loop(0, nc, body, 0)

    @pl.when(cid == 0)
    def _():
        _run(z_a)

    @pl.when(cid == 1)
    def _():
        _run(z_b)

    plsc.subcore_barrier()

    ob = sid * copy_per

    @pl.when(jnp.logical_and(cid == 0, jnp.logical_not(last)))
    def _():
        pltpu.sync_copy(acc.at[pl.ds(ob, copy_per)],
                        out_a.at[pl.ds(ob, copy_per)])

    @pl.when(jnp.logical_and(cid == 0, last))
    def _():
        pltpu.sync_copy(acc.at[pl.ds(ob, copy_last)],
                        out_a.at[pl.ds(ob, copy_last)])

    @pl.when(jnp.logical_and(cid == 1, jnp.logical_not(last)))
    def _():
        pltpu.sync_copy(acc.at[pl.ds(ob, copy_per)],
                        out_b.at[pl.ds(ob, copy_per)])

    @pl.when(jnp.logical_and(cid == 1, last))
    def _():
        pltpu.sync_copy(acc.at[pl.ds(ob, copy_last)],
                        out_b.at[pl.ds(ob, copy_last)])


def _make_segsum(n, d, nc):
    hd = d // 2
    nacc = NS * 8 * (-(-(n + 1) // (NS * 8)))
    mesh = plsc.VectorSubcoreMesh(core_axis_name="c", subcore_axis_name="s")
    return pl.kernel(
        _segsum_body,
        out_type=[jax.ShapeDtypeStruct((n, hd), jnp.float32)] * 2,
        mesh=mesh,
        scratch_types=[
            pltpu.VMEM_SHARED((nacc, hd), jnp.float32),
            pltpu.VMEM((nc, CHUNK), jnp.int32),
            pltpu.VMEM((nc, CHUNK), jnp.int32),
            pltpu.VMEM((CHUNK, hd), jnp.float32),
            pltpu.SemaphoreType.DMA,
        ],
    )


def kernel(x, edge_index, lin_W, lin_b, conv_W, bn_gamma, bn_beta):
    import numpy as np
    n, d = x.shape
    e = edge_index.shape[1]
    nlayers = conv_W.shape[0]

    betas = [float(np.log(THETA / (l + 1) + 1.0)) for l in range(nlayers)]
    eye = jnp.eye(d, dtype=jnp.float32)
    bb = jnp.asarray(betas, jnp.float32).reshape(nlayers, 1, 1)
    Wp = (1.0 - ALPHA) * ((1.0 - bb) * eye[None] + bb * conv_W)

    per = -(-e // NS)
    nc = -(-per // CHUNK)
    src = jnp.pad(edge_index[0], (0, NS * per - e))
    dst = jnp.pad(edge_index[1], (0, NS * per - e), constant_values=n)
    srcp = jnp.pad(src.reshape(NS, per), ((0, 0), (0, nc * CHUNK - per)))
    dstp = jnp.pad(dst.reshape(NS, per), ((0, 0), (0, nc * CHUNK - per)),
                   constant_values=n).reshape(NS, nc, CHUNK)
    srcp = srcp.reshape(NS, nc, CHUNK)

    lin = _make_linear(n, d, ROWS_BLK)
    dense_bn = _make_dense_bn(n, d, ROWS_BLK)
    dense_final = _make_dense_final(n, d, ROWS_BLK)
    segsum = _make_segsum(n, d, nc)

    x0_a, x0_b, xs_a, xs_b = lin(x, lin_W, lin_b.reshape(1, d))
    za, zb = x0_a, x0_b
    for l in range(nlayers):
        h_a, h_b = segsum(za, zb, xs_a, xs_b, srcp, dstp)
        if l < nlayers - 1:
            za, zb = dense_bn(h_a, h_b, Wp[l],
                              bn_gamma[l].reshape(1, d),
                              bn_beta[l].reshape(1, d))
        else:
            z = dense_final(h_a, h_b, Wp[l])
    return z


# speedup vs baseline: 1.4349x; 1.0025x over previous
import functools

import jax
import jax.numpy as jnp
from jax import lax
from jax.experimental import pallas as pl
from jax.experimental.pallas import tpu as pltpu
from jax.experimental.pallas import tpu_sc as plsc

ALPHA = 0.1
THETA = 0.5
EPS = 1e-5

ROWS_BLK = 5000
CHUNK = 128
NS = 16
HP = lax.Precision.HIGHEST


def _linear_body(x_ref, w_ref, b_ref, za_ref, zb_ref, xsa_ref, xsb_ref):
    hd = za_ref.shape[1]
    z = jnp.dot(x_ref[...], w_ref[...], preferred_element_type=jnp.float32,
                precision=HP)
    z = z + b_ref[...]
    za_ref[...] = z[:, :hd]
    zb_ref[...] = z[:, hd:]
    xs = (ALPHA / (1.0 - ALPHA)) * z
    xsa_ref[...] = xs[:, :hd]
    xsb_ref[...] = xs[:, hd:]


def _make_linear(n, d, r):
    nb = n // r
    hd = d // 2
    return pl.pallas_call(
        _linear_body,
        grid=(nb,),
        in_specs=[
            pl.BlockSpec((r, d), lambda j: (j, 0)),
            pl.BlockSpec((d, d), lambda j: (0, 0)),
            pl.BlockSpec((1, d), lambda j: (0, 0)),
        ],
        out_specs=[
            pl.BlockSpec((r, hd), lambda j: (j, 0)),
            pl.BlockSpec((r, hd), lambda j: (j, 0)),
            pl.BlockSpec((r, hd), lambda j: (j, 0)),
            pl.BlockSpec((r, hd), lambda j: (j, 0)),
        ],
        out_shape=[jax.ShapeDtypeStruct((n, hd), jnp.float32)] * 4,
    )


def _dense_bn_body(nb, n_total, h_a, h_b, w_ref, g_ref, be_ref,
                   za_ref, zb_ref, G_s, s_s, aff_s):
    p = pl.program_id(0)
    j = pl.program_id(1)
    hd = za_ref.shape[1]
    hsum = jnp.concatenate([h_a[...], h_b[...]], axis=1)

    @pl.when(jnp.logical_and(p == 0, j == 0))
    def _():
        G_s[...] = jnp.zeros_like(G_s)
        s_s[...] = jnp.zeros_like(s_s)

    @pl.when(p == 0)
    def _():
        G_s[...] += lax.dot_general(hsum, hsum, (((0,), (0,)), ((), ())),
                                    preferred_element_type=jnp.float32,
                                    precision=HP)
        s_s[...] += jnp.sum(hsum, axis=0, keepdims=True)

        @pl.when(j == nb - 1)
        def _():
            W = w_ref[...]
            colsum_z = jnp.dot(s_s[...], W,
                               preferred_element_type=jnp.float32, precision=HP)
            mean = colsum_z / n_total
            GW = jnp.dot(G_s[...], W,
                         preferred_element_type=jnp.float32, precision=HP)
            ez2 = jnp.sum(W * GW, axis=0, keepdims=True) / n_total
            var = ez2 - mean * mean
            scale = g_ref[...] * lax.rsqrt(var + EPS)
            aff_s[0:1, :] = scale
            aff_s[1:2, :] = be_ref[...] - mean * scale

    @pl.when(p == 1)
    def _():
        z = jnp.dot(hsum, w_ref[...], preferred_element_type=jnp.float32,
                    precision=HP)
        zn = jnp.maximum(z * aff_s[0:1, :] + aff_s[1:2, :], 0.0)
        za_ref[...] = zn[:, :hd]
        zb_ref[...] = zn[:, hd:]


def _make_dense_bn(n, d, r):
    nb = n // r
    hd = d // 2
    return pl.pallas_call(
        functools.partial(_dense_bn_body, nb, float(n)),
        grid=(2, nb),
        in_specs=[
            pl.BlockSpec((r, hd), lambda p, j: (j, 0)),
            pl.BlockSpec((r, hd), lambda p, j: (j, 0)),
            pl.BlockSpec((d, d), lambda p, j: (0, 0)),
            pl.BlockSpec((1, d), lambda p, j: (0, 0)),
            pl.BlockSpec((1, d), lambda p, j: (0, 0)),
        ],
        out_specs=[
            pl.BlockSpec((r, hd), lambda p, j: (j, 0)),
            pl.BlockSpec((r, hd), lambda p, j: (j, 0)),
        ],
        out_shape=[jax.ShapeDtypeStruct((n, hd), jnp.float32)] * 2,
        scratch_shapes=[
            pltpu.VMEM((d, d), jnp.float32),
            pltpu.VMEM((1, d), jnp.float32),
            pltpu.VMEM((2, d), jnp.float32),
        ],
    )


def _dense_final_body(h_a, h_b, w_ref, z_ref):
    hsum = jnp.concatenate([h_a[...], h_b[...]], axis=1)
    z_ref[...] = jnp.dot(hsum, w_ref[...], preferred_element_type=jnp.float32,
                         precision=HP)


def _make_dense_final(n, d, r):
    nb = n // r
    hd = d // 2
    return pl.pallas_call(
        _dense_final_body,
        grid=(nb,),
        in_specs=[
            pl.BlockSpec((r, hd), lambda j: (j, 0)),
            pl.BlockSpec((r, hd), lambda j: (j, 0)),
            pl.BlockSpec((d, d), lambda j: (0, 0)),
        ],
        out_specs=pl.BlockSpec((r, d), lambda j: (j, 0)),
        out_shape=jax.ShapeDtypeStruct((n, d), jnp.float32),
    )


def _segsum_body(z_a, z_b, x_a, x_b, srcp, dstp, out_a, out_b,
                 acc, idx_s, idx_d, rows, sem):
    cid = lax.axis_index("c")
    sid = lax.axis_index("s")
    nc = idx_s.shape[0]
    n_out = out_a.shape[0]
    copy_per = 8 * (n_out // (NS * 8))
    copy_last = n_out - copy_per * (NS - 1)
    init_per = copy_per
    init_last = acc.shape[0] - init_per * (NS - 1)

    ib = sid * init_per
    last = sid == NS - 1

    def _init(x_half):
        @pl.when(jnp.logical_not(last))
        def _():
            pltpu.sync_copy(x_half.at[pl.ds(ib, init_per)],
                            acc.at[pl.ds(ib, init_per)])

        @pl.when(last)
        def _():
            pltpu.sync_copy(x_half.at[pl.ds(ib, copy_last)],
                            acc.at[pl.ds(ib, copy_last)])
            pltpu.sync_copy(x_half.at[pl.ds(0, init_last - copy_last)],
                            acc.at[pl.ds(ib + copy_last,
                                         init_last - copy_last)])

    @pl.when(cid == 0)
    def _():
        _init(x_a)

    @pl.when(cid == 1)
    def _():
        _init(x_b)

    pltpu.sync_copy(srcp.at[sid], idx_s)
    pltpu.sync_copy(dstp.at[sid], idx_d)
    plsc.subcore_barrier()

    def _run(table):
        def body(jj, carry):
            pltpu.async_copy(table.at[idx_s.at[jj]], rows, sem).wait()
            pltpu.sync_copy(rows, acc.at[idx_d.at[jj]], add=True)
            return carry
        lax.fori_loop(0, nc, body, 0)

    @pl.when(cid == 0)
    def _():
        _run(z_a)

    @pl.when(cid == 1)
    def _():
        _run(z_b)

    plsc.subcore_barrier()

    ob = sid * copy_per

    @pl.when(jnp.logical_and(cid == 0, jnp.logical_not(last)))
    def _():
        pltpu.sync_copy(acc.at[pl.ds(ob, copy_per)],
                        out_a.at[pl.ds(ob, copy_per)])

    @pl.when(jnp.logical_and(cid == 0, last))
    def _():
        pltpu.sync_copy(acc.at[pl.ds(ob, copy_last)],
                        out_a.at[pl.ds(ob, copy_last)])

    @pl.when(jnp.logical_and(cid == 1, jnp.logical_not(last)))
    def _():
        pltpu.sync_copy(acc.at[pl.ds(ob, copy_per)],
                        out_b.at[pl.ds(ob, copy_per)])

    @pl.when(jnp.logical_and(cid == 1, last))
    def _():
        pltpu.sync_copy(acc.at[pl.ds(ob, copy_last)],
                        out_b.at[pl.ds(ob, copy_last)])


def _make_segsum(n, d, nc):
    hd = d // 2
    nacc = NS * 8 * (-(-(n + 1) // (NS * 8)))
    mesh = plsc.VectorSubcoreMesh(core_axis_name="c", subcore_axis_name="s")
    return pl.kernel(
        _segsum_body,
        out_type=[jax.ShapeDtypeStruct((n, hd), jnp.float32)] * 2,
        mesh=mesh,
        scratch_types=[
            pltpu.VMEM_SHARED((nacc, hd), jnp.float32),
            pltpu.VMEM((nc, CHUNK), jnp.int32),
            pltpu.VMEM((nc, CHUNK), jnp.int32),
            pltpu.VMEM((CHUNK, hd), jnp.float32),
            pltpu.SemaphoreType.DMA,
        ],
    )


def kernel(x, edge_index, lin_W, lin_b, conv_W, bn_gamma, bn_beta):
    import numpy as np
    n, d = x.shape
    e = edge_index.shape[1]
    nlayers = conv_W.shape[0]

    betas = [float(np.log(THETA / (l + 1) + 1.0)) for l in range(nlayers)]
    eye = jnp.eye(d, dtype=jnp.float32)
    bb = jnp.asarray(betas, jnp.float32).reshape(nlayers, 1, 1)
    Wp = (1.0 - ALPHA) * ((1.0 - bb) * eye[None] + bb * conv_W)

    per = -(-e // NS)
    nc = -(-per // CHUNK)
    src = jnp.pad(edge_index[0], (0, NS * per - e))
    dst = jnp.pad(edge_index[1], (0, NS * per - e), constant_values=n)
    srcp = jnp.pad(src.reshape(NS, per), ((0, 0), (0, nc * CHUNK - per)))
    dstp = jnp.pad(dst.reshape(NS, per), ((0, 0), (0, nc * CHUNK - per)),
                   constant_values=n).reshape(NS, nc, CHUNK)
    srcp = srcp.reshape(NS, nc, CHUNK)

    lin = _make_linear(n, d, ROWS_BLK)
    dense_bn = _make_dense_bn(n, d, ROWS_BLK)
    dense_final = _make_dense_final(n, d, ROWS_BLK)
    segsum = _make_segsum(n, d, nc)

    x0_a, x0_b, xs_a, xs_b = lin(x, lin_W, lin_b.reshape(1, d))
    za, zb = x0_a, x0_b
    for l in range(nlayers):
        h_a, h_b = segsum(za, zb, xs_a, xs_b, srcp, dstp)
        if l < nlayers - 1:
            za, zb = dense_bn(h_a, h_b, Wp[l],
                              bn_gamma[l].reshape(1, d),
                              bn_beta[l].reshape(1, d))
        else:
            z = dense_final(h_a, h_b, Wp[l])
    return z
